# baseline (speedup 1.0000x reference)
import functools

import jax
import jax.numpy as jnp
from jax import lax
from jax.experimental import pallas as pl
from jax.experimental.pallas import tpu as pltpu
from jax.experimental.pallas import tpu_sc as plsc

N = 320000
C = 128
T = 8
EPS = 1e-05

NC = 2
NS = 16
NW = NC * NS
R = N // NW
CH = 400
NCHUNK = R // CH
LANES = 16
G = C // LANES

_mesh = plsc.VectorSubcoreMesh(core_axis_name="c", subcore_axis_name="s")


@functools.partial(
    pl.kernel,
    out_type=jax.ShapeDtypeStruct((NW, 2, T, C), jnp.float32),
    mesh=_mesh,
    scratch_types=[
        pltpu.VMEM((CH, C), jnp.float32),
        pltpu.VMEM((CH, C), jnp.float32),
        pltpu.VMEM((16,), jnp.int32),
        pltpu.VMEM((2, T, C), jnp.float32),
        pltpu.SemaphoreType.DMA,
        pltpu.SemaphoreType.DMA,
    ],
)
def _pass1(x_hbm, bnd_hbm, out_hbm, xc0, xc1, bndv, acc, sem0, sem1):
    wid = lax.axis_index("s") * NC + lax.axis_index("c")
    base = wid * R

    bufs = (xc0, xc1)
    sems = (sem0, sem1)

    def _src(k):
        c0 = pl.multiple_of(base + k * CH, 8)
        return x_hbm.at[pl.ds(c0, CH)]

    def _start_in(k, b):
        pltpu.async_copy(_src(k), bufs[b], sems[b])

    def _wait_in(k, b):
        pltpu.make_async_copy(_src(k), bufs[b], sems[b]).wait()

    _start_in(0, 0)
    pltpu.sync_copy(bnd_hbm, bndv)
    bvec = bndv[...]
    bscal = tuple(jnp.clip(bvec[t], base, base + R) for t in range(T + 1))

    zero = jnp.zeros((LANES,), jnp.float32)
    for t in range(T):
        for g in range(G):
            acc[0, t, pl.ds(g * LANES, LANES)] = zero
            acc[1, t, pl.ds(g * LANES, LANES)] = zero

    def _accum_chunk(k, b):
        xc = bufs[b]
        c0 = base + k * CH
        for t in range(T):
            lo = jnp.maximum(bscal[t], c0) - c0
            hi = jnp.minimum(bscal[t + 1], c0 + CH) - c0

            @pl.when(hi > lo)
            def _accum(t=t, lo=lo, hi=hi, xc=xc):
                def body(r, carry):
                    out = []
                    for g in range(G):
                        v = xc[r, pl.ds(g * LANES, LANES)]
                        out.append(carry[g] + v)
                        out.append(carry[G + g] + v * v)
                    return tuple(out[0::2]) + tuple(out[1::2])

                init = (zero,) * (2 * G)
                res = lax.fori_loop(lo, hi, body, init)
                for g in range(G):
                    plsc.addupdate(acc.at[0, t, pl.ds(g * LANES, LANES)],
                                   res[g])
                    plsc.addupdate(acc.at[1, t, pl.ds(g * LANES, LANES)],
                                   res[G + g])

    def _pair(j, carry):
        k0 = 2 * j
        _wait_in(k0, 0)
        _start_in(k0 + 1, 1)
        _accum_chunk(k0, 0)
        _wait_in(k0 + 1, 1)
        _start_in(k0 + 2, 0)
        _accum_chunk(k0 + 1, 1)
        return carry

    lax.fori_loop(0, NCHUNK // 2, _pair, 0)
    _wait_in(NCHUNK - 1, 0)
    _accum_chunk(NCHUNK - 1, 0)
    pltpu.sync_copy(acc, out_hbm.at[wid])


def _bnd_body(tv_ref, out_ref):
    tv = tv_ref[...]
    rows = [jnp.zeros((1, C), jnp.int32)]
    for t in range(1, T + 1):
        rows.append(jnp.sum((tv < t).astype(jnp.int32), axis=0,
                            keepdims=True))
    rows.append(jnp.zeros((16 - (T + 1), C), jnp.int32))
    tot = jnp.sum(jnp.concatenate(rows, axis=0), axis=1, keepdims=True)
    riota = lax.broadcasted_iota(jnp.int32, (16, 1), 0)
    out_ref[...] = jnp.where(riota > T, N, tot)


_bnd = pl.pallas_call(
    _bnd_body,
    out_shape=jax.ShapeDtypeStruct((16, 1), jnp.int32),
)


def _finalize_body(part_ref, bnd_ref, w_ref, b_ref, ss_ref):
    part = part_ref[...]
    s = jnp.sum(part.reshape(NW, 2 * T, C), axis=0)
    sums, sqs = s[:T], s[T:]
    bnd = bnd_ref[...]
    counts = (bnd[1:T + 1] - bnd[:T]).astype(jnp.float32)
    safe = jnp.maximum(counts, 1.0)
    mean = sums / safe
    var = sqs / safe - mean * mean
    inv = lax.rsqrt(jnp.clip(var, EPS, None))
    scale = inv * w_ref[...]
    shift = b_ref[...] - mean * scale
    ss_ref[...] = jnp.concatenate([scale, shift], axis=0)


_finalize = pl.pallas_call(
    _finalize_body,
    out_shape=jax.ShapeDtypeStruct((2 * T, C), jnp.float32),
)

BLK = 1000
NBLK = N // BLK


def _norm_tc_body(ty_ref, ss_ref, x_ref, o_ref):
    ty = ty_ref[0]
    onehot = (ty == lax.broadcasted_iota(jnp.int32, (1, T), 1)
              ).astype(jnp.float32)
    scale = jnp.dot(onehot, ss_ref[:T], preferred_element_type=jnp.float32)
    shift = jnp.dot(onehot, ss_ref[T:], preferred_element_type=jnp.float32)
    o_ref[...] = x_ref[...] * scale + shift


_norm_tc = pl.pallas_call(
    _norm_tc_body,
    grid=(NBLK,),
    in_specs=[
        pl.BlockSpec((1, BLK, 1), lambda i: (i, 0, 0)),
        pl.BlockSpec((2 * T, C), lambda i: (0, 0)),
        pl.BlockSpec((BLK, C), lambda i: (i, 0)),
    ],
    out_specs=pl.BlockSpec((BLK, C), lambda i: (i, 0)),
    out_shape=jax.ShapeDtypeStruct((N, C), jnp.float32),
    compiler_params=pltpu.CompilerParams(
        dimension_semantics=("arbitrary",)),
)


CH2 = 200
NCH2 = R // CH2
NB = 5
NGRP = NCH2 // NB


@functools.partial(
    pl.kernel,
    out_type=jax.ShapeDtypeStruct((N, C), jnp.float32),
    mesh=_mesh,
    scratch_types=[
        pltpu.VMEM((CH2, C), jnp.float32),
        pltpu.VMEM((CH2, C), jnp.float32),
        pltpu.VMEM((CH2, C), jnp.float32),
        pltpu.VMEM((CH2, C), jnp.float32),
        pltpu.VMEM((CH2, C), jnp.float32),
        pltpu.VMEM((16,), jnp.int32),
        pltpu.VMEM((2, T, C), jnp.float32),
        pltpu.SemaphoreType.DMA,
        pltpu.SemaphoreType.DMA,
        pltpu.SemaphoreType.DMA,
        pltpu.SemaphoreType.DMA,
        pltpu.SemaphoreType.DMA,
        pltpu.SemaphoreType.DMA,
        pltpu.SemaphoreType.DMA,
        pltpu.SemaphoreType.DMA,
        pltpu.SemaphoreType.DMA,
        pltpu.SemaphoreType.DMA,
    ],
)
def _pass2(x_hbm, bnd_hbm, ss_hbm, out_hbm, b0, b1, b2, b3, b4, bnd, ss,
           si0, si1, si2, si3, si4, so0, so1, so2, so3, so4):
    wid = lax.axis_index("s") * NC + lax.axis_index("c")
    base = wid * R
    pltpu.sync_copy(bnd_hbm, bnd)
    pltpu.sync_copy(ss_hbm, ss)
    bvec = bnd[...]

    bufs = (b0, b1, b2, b3, b4)
    isems = (si0, si1, si2, si3, si4)
    osems = (so0, so1, so2, so3, so4)

    def _hslice(ref, k):
        c0 = pl.multiple_of(base + k * CH2, 8)
        return ref.at[pl.ds(c0, CH2)]

    def _start_in(k, b):
        pltpu.async_copy(_hslice(x_hbm, k), bufs[b], isems[b])

    def _wait_in(k, b):
        pltpu.make_async_copy(_hslice(x_hbm, k), bufs[b], isems[b]).wait()

    def _start_out(k, b):
        pltpu.async_copy(bufs[b], _hslice(out_hbm, k), osems[b])

    def _wait_out(k, b):
        pltpu.make_async_copy(bufs[b], _hslice(out_hbm, k), osems[b]).wait()

    def _norm_chunk(k, b):
        xc = bufs[b]
        c0 = base + k * CH2
        for t in range(T):
            lo = jnp.maximum(bvec[t], c0) - c0
            hi = jnp.minimum(bvec[t + 1], c0 + CH2) - c0

            @pl.when(hi > lo)
            def _norm(t=t, lo=lo, hi=hi, xc=xc):
                sc = [ss[0, t, pl.ds(g * LANES, LANES)] for g in range(G)]
                sh = [ss[1, t, pl.ds(g * LANES, LANES)] for g in range(G)]

                def body(r, carry):
                    for g in range(G):
                        xc[r, pl.ds(g * LANES, LANES)] = (
                            xc[r, pl.ds(g * LANES, LANES)] * sc[g] + sh[g])
                    return carry

                lax.fori_loop(lo, hi, body, 0)

    for b in range(NB - 1):
        _start_in(b, b)

    def _group(j, carry):
        for b in range(NB):
            k = NB * j + b
            _wait_in(k, b)
            _norm_chunk(k, b)
            _start_out(k, b)
            pb = (b + NB - 1) % NB
            if b == 0:
                @pl.when(j > 0)
                def _(k=k, pb=pb):
                    _wait_out(k - 1, pb)
            else:
                _wait_out(k - 1, pb)

            @pl.when(k + NB - 1 < NCH2)
            def _(k=k, pb=pb):
                _start_in(k + NB - 1, pb)
        return carry

    lax.fori_loop(0, NGRP, _group, 0)

    _wait_out(NCH2 - 1, (NCH2 - 1) % NB)


def kernel(x, weight, bias, type_vec):
    bnd2d = _bnd(type_vec.astype(jnp.int32).reshape(N // C, C))
    bnd16 = bnd2d.reshape(16)
    partials = _pass1(x, bnd16)
    ss = _finalize(partials.reshape(NW * 2 * T, C), bnd16.reshape(16, 1),
                   weight, bias)
    return _pass2(x, bnd16, ss.reshape(2, T, C))


# speedup vs baseline: 1.0019x; 1.0019x over previous
import functools

import jax
import jax.numpy as jnp
from jax import lax
from jax.experimental import pallas as pl
from jax.experimental.pallas import tpu as pltpu
from jax.experimental.pallas import tpu_sc as plsc

N = 320000
C = 128
T = 8
EPS = 1e-05

NC = 2
NS = 16
NW = NC * NS
R = N // NW
CH = 400
NCHUNK = R // CH
LANES = 16
G = C // LANES

_mesh = plsc.VectorSubcoreMesh(core_axis_name="c", subcore_axis_name="s")


@functools.partial(
    pl.kernel,
    out_type=jax.ShapeDtypeStruct((NW, 2, T, C), jnp.float32),
    mesh=_mesh,
    scratch_types=[
        pltpu.VMEM((CH, C), jnp.float32),
        pltpu.VMEM((CH, C), jnp.float32),
        pltpu.VMEM((16,), jnp.int32),
        pltpu.VMEM((2, T, C), jnp.float32),
        pltpu.SemaphoreType.DMA,
        pltpu.SemaphoreType.DMA,
    ],
)
def _pass1(x_hbm, bnd_hbm, out_hbm, xc0, xc1, bndv, acc, sem0, sem1):
    wid = lax.axis_index("s") * NC + lax.axis_index("c")
    base = wid * R

    bufs = (xc0, xc1)
    sems = (sem0, sem1)

    def _src(k):
        c0 = pl.multiple_of(base + k * CH, 8)
        return x_hbm.at[pl.ds(c0, CH)]

    def _start_in(k, b):
        pltpu.async_copy(_src(k), bufs[b], sems[b])

    def _wait_in(k, b):
        pltpu.make_async_copy(_src(k), bufs[b], sems[b]).wait()

    _start_in(0, 0)
    pltpu.sync_copy(bnd_hbm, bndv)
    bvec = bndv[...]
    bscal = tuple(jnp.clip(bvec[t], base, base + R) for t in range(T + 1))

    zero = jnp.zeros((LANES,), jnp.float32)
    for t in range(T):
        for g in range(G):
            acc[0, t, pl.ds(g * LANES, LANES)] = zero
            acc[1, t, pl.ds(g * LANES, LANES)] = zero

    def _accum_chunk(k, b):
        xc = bufs[b]
        c0 = base + k * CH
        for t in range(T):
            lo = jnp.maximum(bscal[t], c0) - c0
            hi = jnp.minimum(bscal[t + 1], c0 + CH) - c0

            @pl.when(hi > lo)
            def _accum(t=t, lo=lo, hi=hi, xc=xc):
                def body(r, carry):
                    out = []
                    for g in range(G):
                        v = xc[r, pl.ds(g * LANES, LANES)]
                        out.append(carry[g] + v)
                        out.append(carry[G + g] + v * v)
                    return tuple(out[0::2]) + tuple(out[1::2])

                init = (zero,) * (2 * G)
                res = lax.fori_loop(lo, hi, body, init)
                for g in range(G):
                    plsc.addupdate(acc.at[0, t, pl.ds(g * LANES, LANES)],
                                   res[g])
                    plsc.addupdate(acc.at[1, t, pl.ds(g * LANES, LANES)],
                                   res[G + g])

    def _pair(j, carry):
        k0 = 2 * j
        _wait_in(k0, 0)
        _start_in(k0 + 1, 1)
        _accum_chunk(k0, 0)
        _wait_in(k0 + 1, 1)
        _start_in(k0 + 2, 0)
        _accum_chunk(k0 + 1, 1)
        return carry

    lax.fori_loop(0, NCHUNK // 2, _pair, 0)
    _wait_in(NCHUNK - 1, 0)
    _accum_chunk(NCHUNK - 1, 0)
    pltpu.sync_copy(acc, out_hbm.at[wid])


def _bnd_body(tv_ref, out_ref):
    tv = tv_ref[...]
    rows = [jnp.zeros((1, C), jnp.int32)]
    for t in range(1, T + 1):
        rows.append(jnp.sum((tv < t).astype(jnp.int32), axis=0,
                            keepdims=True))
    rows.append(jnp.zeros((16 - (T + 1), C), jnp.int32))
    tot = jnp.sum(jnp.concatenate(rows, axis=0), axis=1, keepdims=True)
    riota = lax.broadcasted_iota(jnp.int32, (16, 1), 0)
    out_ref[...] = jnp.where(riota > T, N, tot)


_bnd = pl.pallas_call(
    _bnd_body,
    out_shape=jax.ShapeDtypeStruct((16, 1), jnp.int32),
)


def _finalize_body(part_ref, bnd_ref, w_ref, b_ref, ss_ref):
    part = part_ref[...]
    s = jnp.sum(part.reshape(NW, 2 * T, C), axis=0)
    sums, sqs = s[:T], s[T:]
    bnd = bnd_ref[...]
    counts = (bnd[1:T + 1] - bnd[:T]).astype(jnp.float32)
    safe = jnp.maximum(counts, 1.0)
    mean = sums / safe
    var = sqs / safe - mean * mean
    inv = lax.rsqrt(jnp.clip(var, EPS, None))
    scale = inv * w_ref[...]
    shift = b_ref[...] - mean * scale
    ss_ref[...] = jnp.concatenate([scale, shift], axis=0)


_finalize = pl.pallas_call(
    _finalize_body,
    out_shape=jax.ShapeDtypeStruct((2 * T, C), jnp.float32),
)

CH2 = 200
NCH2 = R // CH2
NB = 5
NGRP = NCH2 // NB


@functools.partial(
    pl.kernel,
    out_type=jax.ShapeDtypeStruct((N, C), jnp.float32),
    mesh=_mesh,
    scratch_types=[
        pltpu.VMEM((CH2, C), jnp.float32),
        pltpu.VMEM((CH2, C), jnp.float32),
        pltpu.VMEM((CH2, C), jnp.float32),
        pltpu.VMEM((CH2, C), jnp.float32),
        pltpu.VMEM((CH2, C), jnp.float32),
        pltpu.VMEM((16,), jnp.int32),
        pltpu.VMEM((2, T, C), jnp.float32),
        pltpu.SemaphoreType.DMA,
        pltpu.SemaphoreType.DMA,
        pltpu.SemaphoreType.DMA,
        pltpu.SemaphoreType.DMA,
        pltpu.SemaphoreType.DMA,
        pltpu.SemaphoreType.DMA,
        pltpu.SemaphoreType.DMA,
        pltpu.SemaphoreType.DMA,
        pltpu.SemaphoreType.DMA,
        pltpu.SemaphoreType.DMA,
    ],
)
def _pass2(x_hbm, bnd_hbm, ss_hbm, out_hbm, b0, b1, b2, b3, b4, bnd, ss,
           si0, si1, si2, si3, si4, so0, so1, so2, so3, so4):
    wid = lax.axis_index("s") * NC + lax.axis_index("c")
    base = wid * R
    pltpu.sync_copy(bnd_hbm, bnd)
    pltpu.sync_copy(ss_hbm, ss)
    bvec = bnd[...]

    bufs = (b0, b1, b2, b3, b4)
    isems = (si0, si1, si2, si3, si4)
    osems = (so0, so1, so2, so3, so4)

    def _hslice(ref, k):
        c0 = pl.multiple_of(base + k * CH2, 8)
        return ref.at[pl.ds(c0, CH2)]

    def _start_in(k, b):
        pltpu.async_copy(_hslice(x_hbm, k), bufs[b], isems[b])

    def _wait_in(k, b):
        pltpu.make_async_copy(_hslice(x_hbm, k), bufs[b], isems[b]).wait()

    def _start_out(k, b):
        pltpu.async_copy(bufs[b], _hslice(out_hbm, k), osems[b])

    def _wait_out(k, b):
        pltpu.make_async_copy(bufs[b], _hslice(out_hbm, k), osems[b]).wait()

    def _norm_chunk(k, b):
        xc = bufs[b]
        c0 = base + k * CH2
        for t in range(T):
            lo = jnp.maximum(bvec[t], c0) - c0
            hi = jnp.minimum(bvec[t + 1], c0 + CH2) - c0

            @pl.when(hi > lo)
            def _norm(t=t, lo=lo, hi=hi, xc=xc):
                sc = [ss[0, t, pl.ds(g * LANES, LANES)] for g in range(G)]
                sh = [ss[1, t, pl.ds(g * LANES, LANES)] for g in range(G)]

                def body(r, carry):
                    for g in range(G):
                        xc[r, pl.ds(g * LANES, LANES)] = (
                            xc[r, pl.ds(g * LANES, LANES)] * sc[g] + sh[g])
                    return carry

                lax.fori_loop(lo, hi, body, 0)

    for b in range(NB - 1):
        _start_in(b, b)

    def _group(j, carry):
        for b in range(NB):
            k = NB * j + b
            _wait_in(k, b)
            _norm_chunk(k, b)
            _start_out(k, b)
            pb = (b + NB - 1) % NB
            if b == 0:
                @pl.when(j > 0)
                def _(k=k, pb=pb):
                    _wait_out(k - 1, pb)
            else:
                _wait_out(k - 1, pb)

            @pl.when(k + NB - 1 < NCH2)
            def _(k=k, pb=pb):
                _start_in(k + NB - 1, pb)
        return carry

    lax.fori_loop(0, NGRP, _group, 0)

    _wait_out(NCH2 - 1, (NCH2 - 1) % NB)


def kernel(x, weight, bias, type_vec):
    bnd2d = _bnd(type_vec.astype(jnp.int32).reshape(N // C, C))
    bnd16 = bnd2d.reshape(16)
    partials = _pass1(x, bnd16)
    ss = _finalize(partials.reshape(NW * 2 * T, C), bnd16.reshape(16, 1),
                   weight, bias)
    return _pass2(x, bnd16, ss.reshape(2, T, C))
